# baseline (speedup 1.0000x reference)
import jax
import jax.numpy as jnp
from jax import lax
from jax.experimental import pallas as pl
from jax.experimental.pallas import tpu as pltpu
from jax.experimental.pallas import tpu_sc as plsc

N = 10000
E = 320000
D = 128
R = 3

NC = 2
NS = 16
NW = NC * NS
EPW = E // NW
K = 128
NFULL = EPW // K
TAIL = EPW - NFULL * K
CNT = R * N
CNT_PAD = 30720
CPT = CNT_PAD // NS
CH = 40
NCHW = EPW // CH
NSLOT = 5
NZC = N // CH
STG = EPW + 64

_mesh = plsc.VectorSubcoreMesh(core_axis_name="c", subcore_axis_name="s")


def _worker():
    cid = lax.axis_index("c")
    sid = lax.axis_index("s")
    return cid, sid, sid * NC + cid


def _pass_a_body(src_hbm, dst_hbm, typ_hbm, cnt2_hbm, lsrc_hbm, ldst_hbm,
                 ncnt_hbm, src_v, dst_v, typ_v, c_v, ones_v, zb_v,
                 s16, d16, t16, c16, o16, nv_v, stg_s, stg_d, cnt_sh):
    cid, sid, wid = _worker()
    base = wid * EPW
    iota = jnp.arange(16, dtype=jnp.int32)

    @pl.loop(0, CPT // 16)
    def _zero_zb(i):
        zb_v[pl.ds(i * 16, 16)] = jnp.zeros((16,), jnp.float32)

    @pl.loop(0, K // 16)
    def _ones(i):
        ones_v[pl.ds(i * 16, 16)] = jnp.full((16,), 1.0, jnp.float32)

    o16[pl.ds(0, 16)] = jnp.full((16,), 1.0, jnp.float32)

    pltpu.sync_copy(zb_v, cnt_sh.at[pl.ds(sid * CPT, CPT)])
    plsc.subcore_barrier()

    def group(sl, sv_ref, dv_ref, tv_ref, cv_ref, offs):
        tv = tv_ref[sl]
        sv = sv_ref[sl]
        dv = dv_ref[sl]
        cv_ref[sl] = tv * N + dv
        new_offs = []
        for r in range(R):
            mi = (tv == r).astype(jnp.int32)
            pos = offs[r] + plsc.cumsum(mi) - mi
            m = mi != 0
            plsc.store_scatter(stg_s[r], [pos], sv, mask=m)
            plsc.store_scatter(stg_d[r], [pos], dv, mask=m)
            new_offs.append(offs[r] + jnp.sum(mi))
        return tuple(new_offs)

    @pl.loop(0, NFULL, init_carry=(0, 0, 0))
    def _chunks(i, offs):
        off = base + i * K
        pltpu.sync_copy(src_hbm.at[pl.ds(off, K)], src_v)
        pltpu.sync_copy(dst_hbm.at[pl.ds(off, K)], dst_v)
        pltpu.sync_copy(typ_hbm.at[pl.ds(off, K)], typ_v)
        for j in range(K // 16):
            offs = group(pl.ds(j * 16, 16), src_v, dst_v, typ_v, c_v, offs)
        pltpu.sync_copy(ones_v, cnt_sh.at[c_v], add=True)
        return offs

    toff = base + NFULL * K
    pltpu.sync_copy(src_hbm.at[pl.ds(toff, TAIL)], s16)
    pltpu.sync_copy(dst_hbm.at[pl.ds(toff, TAIL)], d16)
    pltpu.sync_copy(typ_hbm.at[pl.ds(toff, TAIL)], t16)
    offs = group(pl.ds(0, 16), s16, d16, t16, c16, _chunks)
    pltpu.sync_copy(o16, cnt_sh.at[c16], add=True)

    nchs = []
    for r in range(R):
        for kk in range(3):
            pos = offs[r] + kk * 16 + iota
            plsc.store_scatter(stg_s[r], [pos],
                               jnp.zeros((16,), jnp.int32))
            plsc.store_scatter(stg_d[r], [pos],
                               jnp.full((16,), N, jnp.int32))
        nch = (offs[r] + (CH - 1)) // CH
        nchs.append(nch)

        lbase = (r * NW + wid) * EPW

        @pl.loop(0, nch)
        def _dma_out(i):
            o = pl.multiple_of(i * CH, CH)
            pltpu.sync_copy(stg_s[r].at[pl.ds(o, CH)],
                            lsrc_hbm.at[pl.ds(lbase + o, CH)])
            pltpu.sync_copy(stg_d[r].at[pl.ds(o, CH)],
                            ldst_hbm.at[pl.ds(lbase + o, CH)])

    cvec = (jnp.where(iota == 0, nchs[0], 0)
            + jnp.where(iota == 1, nchs[1], 0)
            + jnp.where(iota == 2, nchs[2], 0)).astype(jnp.int32)
    nv_v[pl.ds(0, 16)] = cvec
    pltpu.sync_copy(nv_v, ncnt_hbm.at[wid, 0])

    plsc.subcore_barrier()
    pltpu.sync_copy(cnt_sh.at[pl.ds(sid * CPT, CPT)],
                    cnt2_hbm.at[cid, 0, pl.ds(sid * CPT, CPT)])


_pass_a = pl.kernel(
    _pass_a_body,
    out_type=(
        jax.ShapeDtypeStruct((NC, 1, CNT_PAD), jnp.float32),
        jax.ShapeDtypeStruct((R * NW * EPW,), jnp.int32),
        jax.ShapeDtypeStruct((R * NW * EPW,), jnp.int32),
        jax.ShapeDtypeStruct((NW, 1, 16), jnp.int32),
    ),
    mesh=_mesh,
    compiler_params=pltpu.CompilerParams(needs_layout_passes=False),
    scratch_types=[
        pltpu.VMEM((K,), jnp.int32),
        pltpu.VMEM((K,), jnp.int32),
        pltpu.VMEM((K,), jnp.int32),
        pltpu.VMEM((K,), jnp.int32),
        pltpu.VMEM((K,), jnp.float32),
        pltpu.VMEM((CPT,), jnp.float32),
        pltpu.VMEM((16,), jnp.int32),
        pltpu.VMEM((16,), jnp.int32),
        pltpu.VMEM((16,), jnp.int32),
        pltpu.VMEM((16,), jnp.int32),
        pltpu.VMEM((16,), jnp.float32),
        pltpu.VMEM((16,), jnp.int32),
        [pltpu.VMEM((STG,), jnp.int32) for _ in range(R)],
        [pltpu.VMEM((STG,), jnp.int32) for _ in range(R)],
        pltpu.VMEM_SHARED((CNT_PAD,), jnp.float32),
    ],
)


IPW = CNT_PAD // NS


def _pass_b_body(cnt2_hbm, inv_hbm, a_v, b_v, inv_v):
    cid, sid, wid = _worker()

    @pl.when(cid == 0)
    def _go():
        off = sid * IPW
        pltpu.sync_copy(cnt2_hbm.at[0, 0, pl.ds(off, IPW)], a_v)
        pltpu.sync_copy(cnt2_hbm.at[1, 0, pl.ds(off, IPW)], b_v)

        @pl.loop(0, IPW // 16)
        def _inv(i):
            sl = pl.ds(i * 16, 16)
            inv_v[sl] = 1.0 / jnp.maximum(a_v[sl] + b_v[sl], 1.0)

        pltpu.sync_copy(inv_v, inv_hbm.at[pl.ds(off, IPW)])


_pass_b = pl.kernel(
    _pass_b_body,
    out_type=jax.ShapeDtypeStruct((CNT_PAD,), jnp.float32),
    mesh=_mesh,
    compiler_params=pltpu.CompilerParams(needs_layout_passes=False),
    scratch_types=[
        pltpu.VMEM((IPW,), jnp.float32),
        pltpu.VMEM((IPW,), jnp.float32),
        pltpu.VMEM((IPW,), jnp.float32),
    ],
)


def _pass_m_body(h_hbm, lsrc_hbm, ldst_hbm, ncnt_hbm, acc_hbm,
                 gbufs, dbufs, rows, zrow, cbuf,
                 gsems, isems, ssems, acc_sh):
    cid, sid, wid = _worker()
    iota = jnp.arange(16, dtype=jnp.int32)

    pltpu.sync_copy(ncnt_hbm.at[wid, 0], cbuf)
    nch_vec = cbuf[pl.ds(0, 16)]

    @pl.loop(0, CH)
    def _zero_zrow(i):
        for j in range(D // 16):
            zrow[i, pl.ds(j * 16, 16)] = jnp.zeros((16,), jnp.float32)

    for r in range(R):
        nv = jnp.max(jnp.where(iota == r, nch_vec, 0))

        lbase = (r * NW + wid) * EPW

        def idx_start(c, s):
            off = pl.ds(lbase + pl.multiple_of(c * CH, CH), CH)
            pltpu.async_copy(lsrc_hbm.at[off], gbufs[s], isems[s])
            pltpu.async_copy(ldst_hbm.at[off], dbufs[s], isems[s])

        def idx_wait(c, s):
            off = pl.ds(lbase + pl.multiple_of(c * CH, CH), CH)
            pltpu.make_async_copy(lsrc_hbm.at[off], gbufs[s],
                                  isems[s]).wait()
            pltpu.make_async_copy(ldst_hbm.at[off], dbufs[s],
                                  isems[s]).wait()

        def scat_wait(s):
            pltpu.make_async_copy(rows[s], acc_sh.at[dbufs[s]],
                                  ssems[s]).wait()

        for k in range(16):
            zc = sid + NS * k

            @pl.when(zc < NZC)
            def _zero_chunk():
                off = pl.multiple_of(zc * CH, CH)
                pltpu.sync_copy(zrow, acc_sh.at[pl.ds(off, CH)])

        plsc.subcore_barrier()

        for c in range(3):

            @pl.when(c < nv)
            def _prime_idx():
                idx_start(c, c)

        for c in range(2):

            @pl.when(c < nv)
            def _prime_gather():
                idx_wait(c, c)
                pltpu.async_copy(h_hbm.at[gbufs[c]], rows[c], gsems[c])

        @pl.loop(0, NCHW // NSLOT)
        def _visits(i):
            for b in range(NSLOT):
                c = i * NSLOT + b

                s3 = (b + 3) % NSLOT
                c3 = c + 3

                @pl.when(c3 < nv)
                def _issue_idx():
                    @pl.when(c3 >= NSLOT)
                    def _drain_scat():
                        scat_wait(s3)

                    idx_start(c3, s3)

                s2 = (b + 2) % NSLOT
                c2 = c + 2

                @pl.when(c2 < nv)
                def _issue_gather():
                    idx_wait(c2, s2)
                    pltpu.async_copy(h_hbm.at[gbufs[s2]], rows[s2],
                                     gsems[s2])

                @pl.when(c < nv)
                def _process():
                    pltpu.make_async_copy(h_hbm.at[gbufs[b]], rows[b],
                                          gsems[b]).wait()
                    pltpu.async_copy(rows[b], acc_sh.at[dbufs[b]],
                                     ssems[b], add=True)

        for s in range(NSLOT):

            @pl.when(s < nv)
            def _final_drain():
                scat_wait(s)

        plsc.subcore_barrier()
        for k in range(16):
            zc = sid + NS * k

            @pl.when(zc < NZC)
            def _dump_chunk():
                off = pl.multiple_of(zc * CH, CH)
                pltpu.sync_copy(acc_sh.at[pl.ds(off, CH)],
                                acc_hbm.at[r, cid, pl.ds(off, CH)])

        plsc.subcore_barrier()


_pass_m = pl.kernel(
    _pass_m_body,
    out_type=jax.ShapeDtypeStruct((R, NC, N, D), jnp.float32),
    mesh=_mesh,
    compiler_params=pltpu.CompilerParams(needs_layout_passes=False),
    scratch_types=[
        [pltpu.VMEM((CH,), jnp.int32) for _ in range(NSLOT)],
        [pltpu.VMEM((CH,), jnp.int32) for _ in range(NSLOT)],
        [pltpu.VMEM((CH, D), jnp.float32) for _ in range(NSLOT)],
        pltpu.VMEM((CH, D), jnp.float32),
        pltpu.VMEM((16,), jnp.int32),
        [pltpu.SemaphoreType.DMA for _ in range(NSLOT)],
        [pltpu.SemaphoreType.DMA for _ in range(NSLOT)],
        [pltpu.SemaphoreType.DMA for _ in range(NSLOT)],
        pltpu.VMEM_SHARED((N + 8, D), jnp.float32),
    ],
)


BLK = 1000
NB = N // BLK


def _mm(a, b):
    return jnp.dot(a, b, preferred_element_type=jnp.float32)


def _k_in_body(x_ref, win_ref, bin_ref, wroot_ref, h_out, root_out):
    h = jnp.maximum(_mm(x_ref[...], win_ref[...]) + bin_ref[0][None, :], 0.0)
    h_out[...] = h
    root_out[...] = _mm(h, wroot_ref[...])


_k_in = pl.pallas_call(
    _k_in_body,
    grid=(NB,),
    in_specs=[
        pl.BlockSpec((BLK, D), lambda i: (i, 0)),
        pl.BlockSpec((D, D), lambda i: (0, 0)),
        pl.BlockSpec((1, D), lambda i: (0, 0)),
        pl.BlockSpec((D, D), lambda i: (0, 0)),
    ],
    out_specs=[
        pl.BlockSpec((BLK, D), lambda i: (i, 0)),
        pl.BlockSpec((BLK, D), lambda i: (i, 0)),
    ],
    out_shape=[
        jax.ShapeDtypeStruct((N, D), jnp.float32),
        jax.ShapeDtypeStruct((N, D), jnp.float32),
    ],
)


def _agg_sum(root_ref, b_ref, acc_ref, inv_ref, wrel_ref):
    t = root_ref[...] + b_ref[0][None, :]
    for r in range(R):
        agg = (acc_ref[r, 0] + acc_ref[r, 1]) * inv_ref[r]
        t = t + _mm(agg, wrel_ref[r])
    return jnp.maximum(t, 0.0)


def _k_mid_body(root_ref, b_ref, acc_ref, inv_ref, wrel_ref, wrootn_ref,
                h_out, root_out):
    h = _agg_sum(root_ref, b_ref, acc_ref, inv_ref, wrel_ref)
    h_out[...] = h
    root_out[...] = _mm(h, wrootn_ref[...])


_k_mid = pl.pallas_call(
    _k_mid_body,
    grid=(NB,),
    in_specs=[
        pl.BlockSpec((BLK, D), lambda i: (i, 0)),
        pl.BlockSpec((1, D), lambda i: (0, 0)),
        pl.BlockSpec((R, NC, BLK, D), lambda i: (0, 0, i, 0)),
        pl.BlockSpec((R, BLK, 1), lambda i: (0, i, 0)),
        pl.BlockSpec((R, D, D), lambda i: (0, 0, 0)),
        pl.BlockSpec((D, D), lambda i: (0, 0)),
    ],
    out_specs=[
        pl.BlockSpec((BLK, D), lambda i: (i, 0)),
        pl.BlockSpec((BLK, D), lambda i: (i, 0)),
    ],
    out_shape=[
        jax.ShapeDtypeStruct((N, D), jnp.float32),
        jax.ShapeDtypeStruct((N, D), jnp.float32),
    ],
)


def _k_mlp_body(root_ref, b_ref, acc_ref, inv_ref, wrel_ref,
                wo1_ref, bo1_ref, wo2_ref, bo2_ref, wo3_ref, bo3_ref,
                out_ref):
    h = _agg_sum(root_ref, b_ref, acc_ref, inv_ref, wrel_ref)
    o = jnp.maximum(_mm(h, wo1_ref[...]) + bo1_ref[0][None, :], 0.0)
    o = jnp.maximum(_mm(o, wo2_ref[...]) + bo2_ref[0][None, :], 0.0)
    out_ref[...] = _mm(o, wo3_ref[...]) + bo3_ref[0][None, :]


_k_mlp = pl.pallas_call(
    _k_mlp_body,
    grid=(NB,),
    in_specs=[
        pl.BlockSpec((BLK, D), lambda i: (i, 0)),
        pl.BlockSpec((1, D), lambda i: (0, 0)),
        pl.BlockSpec((R, NC, BLK, D), lambda i: (0, 0, i, 0)),
        pl.BlockSpec((R, BLK, 1), lambda i: (0, i, 0)),
        pl.BlockSpec((R, D, D), lambda i: (0, 0, 0)),
        pl.BlockSpec((D, 512), lambda i: (0, 0)),
        pl.BlockSpec((1, 512), lambda i: (0, 0)),
        pl.BlockSpec((512, 256), lambda i: (0, 0)),
        pl.BlockSpec((1, 256), lambda i: (0, 0)),
        pl.BlockSpec((256, 128), lambda i: (0, 0)),
        pl.BlockSpec((1, 128), lambda i: (0, 0)),
    ],
    out_specs=pl.BlockSpec((BLK, 128), lambda i: (i, 0)),
    out_shape=jax.ShapeDtypeStruct((N, 128), jnp.float32),
)


def kernel(x, edge_index, edge_type, W_in, b_in, w1_rel, w1_root, b1,
           w2_rel, w2_root, b2, w3_rel, w3_root, b3,
           Wo1, bo1, Wo2, bo2, Wo3, bo3):
    src = edge_index[0]
    dst = edge_index[1]

    cnt2, lsrc, ldst, ncnt = _pass_a(src, dst, edge_type)
    inv3 = _pass_b(cnt2)[:CNT].reshape(R, N, 1)

    h, root = _k_in(x, W_in, b_in.reshape(1, D), w1_root)
    acc = _pass_m(h, lsrc, ldst, ncnt)
    h, root = _k_mid(root, b1.reshape(1, D), acc, inv3, w1_rel, w2_root)
    acc = _pass_m(h, lsrc, ldst, ncnt)
    h, root = _k_mid(root, b2.reshape(1, D), acc, inv3, w2_rel, w3_root)
    acc = _pass_m(h, lsrc, ldst, ncnt)

    wo3p = jnp.pad(Wo3, ((0, 0), (0, 128 - Wo3.shape[1])))
    bo3p = jnp.pad(bo3, (0, 128 - bo3.shape[0]))
    out = _k_mlp(root, b3.reshape(1, D), acc, inv3, w3_rel,
                 Wo1, bo1.reshape(1, 512), Wo2, bo2.reshape(1, 256),
                 wo3p, bo3p.reshape(1, 128))
    return out[:, :Wo3.shape[1]]


# speedup vs baseline: 1.0385x; 1.0385x over previous
import jax
import jax.numpy as jnp
from jax import lax
from jax.experimental import pallas as pl
from jax.experimental.pallas import tpu as pltpu
from jax.experimental.pallas import tpu_sc as plsc

N = 10000
E = 320000
D = 128
R = 3

NC = 2
NS = 16
NW = NC * NS
EPW = E // NW
K = 128
NFULL = EPW // K
TAIL = EPW - NFULL * K
CNT = R * N
CNT_PAD = 30720
CPT = CNT_PAD // NS
CH = 40
NCHW = EPW // CH
NSLOT = 5
NZC = N // CH
CB = 320
RSEG = ((EPW + CB - 1) // CB + 1) * CB
STG = RSEG

_mesh = plsc.VectorSubcoreMesh(core_axis_name="c", subcore_axis_name="s")


def _worker():
    cid = lax.axis_index("c")
    sid = lax.axis_index("s")
    return cid, sid, sid * NC + cid


def _pass_a_body(src_hbm, dst_hbm, typ_hbm, cnt2_hbm, lsrc_hbm, ldst_hbm,
                 ncnt_hbm, src_v, dst_v, typ_v, c_v, ones_v, zb_v,
                 s16, d16, t16, c16, o16, nv_v, stg_s, stg_d, cnt_sh):
    cid, sid, wid = _worker()
    base = wid * EPW
    iota = jnp.arange(16, dtype=jnp.int32)

    @pl.loop(0, CPT // 16)
    def _zero_zb(i):
        zb_v[pl.ds(i * 16, 16)] = jnp.zeros((16,), jnp.float32)

    @pl.loop(0, K // 16)
    def _ones(i):
        ones_v[pl.ds(i * 16, 16)] = jnp.full((16,), 1.0, jnp.float32)

    o16[pl.ds(0, 16)] = jnp.full((16,), 1.0, jnp.float32)

    pltpu.sync_copy(zb_v, cnt_sh.at[pl.ds(sid * CPT, CPT)])
    plsc.subcore_barrier()

    def group(sl, sv_ref, dv_ref, tv_ref, cv_ref, offs):
        tv = tv_ref[sl]
        sv = sv_ref[sl]
        dv = dv_ref[sl]
        cv_ref[sl] = tv * N + dv
        new_offs = []
        for r in range(R):
            mi = (tv == r).astype(jnp.int32)
            pos = offs[r] + plsc.cumsum(mi) - mi
            m = mi != 0
            plsc.store_scatter(stg_s[r], [pos], sv, mask=m)
            plsc.store_scatter(stg_d[r], [pos], dv, mask=m)
            new_offs.append(offs[r] + jnp.sum(mi))
        return tuple(new_offs)

    @pl.loop(0, NFULL, init_carry=(0, 0, 0))
    def _chunks(i, offs):
        off = base + i * K
        pltpu.sync_copy(src_hbm.at[pl.ds(off, K)], src_v)
        pltpu.sync_copy(dst_hbm.at[pl.ds(off, K)], dst_v)
        pltpu.sync_copy(typ_hbm.at[pl.ds(off, K)], typ_v)
        for j in range(K // 16):
            offs = group(pl.ds(j * 16, 16), src_v, dst_v, typ_v, c_v, offs)
        pltpu.sync_copy(ones_v, cnt_sh.at[c_v], add=True)
        return offs

    toff = base + NFULL * K
    pltpu.sync_copy(src_hbm.at[pl.ds(toff, TAIL)], s16)
    pltpu.sync_copy(dst_hbm.at[pl.ds(toff, TAIL)], d16)
    pltpu.sync_copy(typ_hbm.at[pl.ds(toff, TAIL)], t16)
    offs = group(pl.ds(0, 16), s16, d16, t16, c16, _chunks)
    pltpu.sync_copy(o16, cnt_sh.at[c16], add=True)

    nchs = []
    for r in range(R):
        for kk in range(3):
            pos = offs[r] + kk * 16 + iota
            plsc.store_scatter(stg_s[r], [pos],
                               jnp.zeros((16,), jnp.int32))
            plsc.store_scatter(stg_d[r], [pos],
                               jnp.full((16,), N, jnp.int32))
        nch = (offs[r] + (CH - 1)) // CH
        nchs.append(nch)

        lbase = (r * NW + wid) * RSEG
        nbig = (offs[r] + (CB - 1)) // CB

        @pl.loop(0, nbig)
        def _dma_out(i):
            o = pl.multiple_of(i * CB, CB)
            pltpu.sync_copy(stg_s[r].at[pl.ds(o, CB)],
                            lsrc_hbm.at[pl.ds(lbase + o, CB)])
            pltpu.sync_copy(stg_d[r].at[pl.ds(o, CB)],
                            ldst_hbm.at[pl.ds(lbase + o, CB)])

    cvec = (jnp.where(iota == 0, nchs[0], 0)
            + jnp.where(iota == 1, nchs[1], 0)
            + jnp.where(iota == 2, nchs[2], 0)).astype(jnp.int32)
    nv_v[pl.ds(0, 16)] = cvec
    pltpu.sync_copy(nv_v, ncnt_hbm.at[wid, 0])

    plsc.subcore_barrier()
    pltpu.sync_copy(cnt_sh.at[pl.ds(sid * CPT, CPT)],
                    cnt2_hbm.at[cid, 0, pl.ds(sid * CPT, CPT)])


_pass_a = pl.kernel(
    _pass_a_body,
    out_type=(
        jax.ShapeDtypeStruct((NC, 1, CNT_PAD), jnp.float32),
        jax.ShapeDtypeStruct((R * NW * RSEG,), jnp.int32),
        jax.ShapeDtypeStruct((R * NW * RSEG,), jnp.int32),
        jax.ShapeDtypeStruct((NW, 1, 16), jnp.int32),
    ),
    mesh=_mesh,
    compiler_params=pltpu.CompilerParams(needs_layout_passes=False),
    scratch_types=[
        pltpu.VMEM((K,), jnp.int32),
        pltpu.VMEM((K,), jnp.int32),
        pltpu.VMEM((K,), jnp.int32),
        pltpu.VMEM((K,), jnp.int32),
        pltpu.VMEM((K,), jnp.float32),
        pltpu.VMEM((CPT,), jnp.float32),
        pltpu.VMEM((16,), jnp.int32),
        pltpu.VMEM((16,), jnp.int32),
        pltpu.VMEM((16,), jnp.int32),
        pltpu.VMEM((16,), jnp.int32),
        pltpu.VMEM((16,), jnp.float32),
        pltpu.VMEM((16,), jnp.int32),
        [pltpu.VMEM((STG,), jnp.int32) for _ in range(R)],
        [pltpu.VMEM((STG,), jnp.int32) for _ in range(R)],
        pltpu.VMEM_SHARED((CNT_PAD,), jnp.float32),
    ],
)


IPW = CNT_PAD // NS


def _pass_b_body(cnt2_hbm, inv_hbm, a_v, b_v, inv_v):
    cid, sid, wid = _worker()

    @pl.when(cid == 0)
    def _go():
        off = sid * IPW
        pltpu.sync_copy(cnt2_hbm.at[0, 0, pl.ds(off, IPW)], a_v)
        pltpu.sync_copy(cnt2_hbm.at[1, 0, pl.ds(off, IPW)], b_v)

        @pl.loop(0, IPW // 16)
        def _inv(i):
            sl = pl.ds(i * 16, 16)
            inv_v[sl] = 1.0 / jnp.maximum(a_v[sl] + b_v[sl], 1.0)

        pltpu.sync_copy(inv_v, inv_hbm.at[pl.ds(off, IPW)])


_pass_b = pl.kernel(
    _pass_b_body,
    out_type=jax.ShapeDtypeStruct((CNT_PAD,), jnp.float32),
    mesh=_mesh,
    compiler_params=pltpu.CompilerParams(needs_layout_passes=False),
    scratch_types=[
        pltpu.VMEM((IPW,), jnp.float32),
        pltpu.VMEM((IPW,), jnp.float32),
        pltpu.VMEM((IPW,), jnp.float32),
    ],
)


def _pass_m_body(h_hbm, lsrc_hbm, ldst_hbm, ncnt_hbm, acc_hbm,
                 gbufs, dbufs, rows, zrow, cbuf,
                 gsems, isems, ssems, acc_sh):
    cid, sid, wid = _worker()
    iota = jnp.arange(16, dtype=jnp.int32)

    pltpu.sync_copy(ncnt_hbm.at[wid, 0], cbuf)
    nch_vec = cbuf[pl.ds(0, 16)]

    @pl.loop(0, CH)
    def _zero_zrow(i):
        for j in range(D // 16):
            zrow[i, pl.ds(j * 16, 16)] = jnp.zeros((16,), jnp.float32)

    for r in range(R):
        nv = jnp.max(jnp.where(iota == r, nch_vec, 0))

        lbase = (r * NW + wid) * RSEG

        def idx_start(c, s):
            off = pl.ds(lbase + pl.multiple_of(c * CH, CH), CH)
            pltpu.async_copy(lsrc_hbm.at[off], gbufs[s], isems[s])
            pltpu.async_copy(ldst_hbm.at[off], dbufs[s], isems[s])

        def idx_wait(c, s):
            off = pl.ds(lbase + pl.multiple_of(c * CH, CH), CH)
            pltpu.make_async_copy(lsrc_hbm.at[off], gbufs[s],
                                  isems[s]).wait()
            pltpu.make_async_copy(ldst_hbm.at[off], dbufs[s],
                                  isems[s]).wait()

        def scat_wait(s):
            pltpu.make_async_copy(rows[s], acc_sh.at[dbufs[s]],
                                  ssems[s]).wait()

        for k in range(16):
            zc = sid + NS * k

            @pl.when(zc < NZC)
            def _zero_chunk():
                off = pl.multiple_of(zc * CH, CH)
                pltpu.sync_copy(zrow, acc_sh.at[pl.ds(off, CH)])

        plsc.subcore_barrier()

        for c in range(3):

            @pl.when(c < nv)
            def _prime_idx():
                idx_start(c, c)

        for c in range(2):

            @pl.when(c < nv)
            def _prime_gather():
                idx_wait(c, c)
                pltpu.async_copy(h_hbm.at[gbufs[c]], rows[c], gsems[c])

        nvis = (nv + (NSLOT - 1)) // NSLOT

        @pl.loop(0, nvis)
        def _visits(i):
            for b in range(NSLOT):
                c = i * NSLOT + b

                s3 = (b + 3) % NSLOT
                c3 = c + 3

                @pl.when(c3 < nv)
                def _issue_idx():
                    @pl.when(c3 >= NSLOT)
                    def _drain_scat():
                        scat_wait(s3)

                    idx_start(c3, s3)

                s2 = (b + 2) % NSLOT
                c2 = c + 2

                @pl.when(c2 < nv)
                def _issue_gather():
                    idx_wait(c2, s2)
                    pltpu.async_copy(h_hbm.at[gbufs[s2]], rows[s2],
                                     gsems[s2])

                @pl.when(c < nv)
                def _process():
                    pltpu.make_async_copy(h_hbm.at[gbufs[b]], rows[b],
                                          gsems[b]).wait()
                    pltpu.async_copy(rows[b], acc_sh.at[dbufs[b]],
                                     ssems[b], add=True)

        for s in range(NSLOT):

            @pl.when(s < nv)
            def _final_drain():
                scat_wait(s)

        plsc.subcore_barrier()
        for k in range(16):
            zc = sid + NS * k

            @pl.when(zc < NZC)
            def _dump_chunk():
                off = pl.multiple_of(zc * CH, CH)
                pltpu.sync_copy(acc_sh.at[pl.ds(off, CH)],
                                acc_hbm.at[r, cid, pl.ds(off, CH)])

        plsc.subcore_barrier()


_pass_m = pl.kernel(
    _pass_m_body,
    out_type=jax.ShapeDtypeStruct((R, NC, N, D), jnp.float32),
    mesh=_mesh,
    compiler_params=pltpu.CompilerParams(needs_layout_passes=False),
    scratch_types=[
        [pltpu.VMEM((CH,), jnp.int32) for _ in range(NSLOT)],
        [pltpu.VMEM((CH,), jnp.int32) for _ in range(NSLOT)],
        [pltpu.VMEM((CH, D), jnp.float32) for _ in range(NSLOT)],
        pltpu.VMEM((CH, D), jnp.float32),
        pltpu.VMEM((16,), jnp.int32),
        [pltpu.SemaphoreType.DMA for _ in range(NSLOT)],
        [pltpu.SemaphoreType.DMA for _ in range(NSLOT)],
        [pltpu.SemaphoreType.DMA for _ in range(NSLOT)],
        pltpu.VMEM_SHARED((N + 8, D), jnp.float32),
    ],
)


BLK = 1000
NB = N // BLK


def _mm(a, b):
    return jnp.dot(a, b, preferred_element_type=jnp.float32)


def _k_in_body(x_ref, win_ref, bin_ref, wroot_ref, h_out, root_out):
    h = jnp.maximum(_mm(x_ref[...], win_ref[...]) + bin_ref[0][None, :], 0.0)
    h_out[...] = h
    root_out[...] = _mm(h, wroot_ref[...])


_k_in = pl.pallas_call(
    _k_in_body,
    grid=(NB,),
    in_specs=[
        pl.BlockSpec((BLK, D), lambda i: (i, 0)),
        pl.BlockSpec((D, D), lambda i: (0, 0)),
        pl.BlockSpec((1, D), lambda i: (0, 0)),
        pl.BlockSpec((D, D), lambda i: (0, 0)),
    ],
    out_specs=[
        pl.BlockSpec((BLK, D), lambda i: (i, 0)),
        pl.BlockSpec((BLK, D), lambda i: (i, 0)),
    ],
    out_shape=[
        jax.ShapeDtypeStruct((N, D), jnp.float32),
        jax.ShapeDtypeStruct((N, D), jnp.float32),
    ],
)


def _agg_sum(root_ref, b_ref, acc_ref, inv_ref, wrel_ref):
    t = root_ref[...] + b_ref[0][None, :]
    for r in range(R):
        agg = (acc_ref[r, 0] + acc_ref[r, 1]) * inv_ref[r]
        t = t + _mm(agg, wrel_ref[r])
    return jnp.maximum(t, 0.0)


def _k_mid_body(root_ref, b_ref, acc_ref, inv_ref, wrel_ref, wrootn_ref,
                h_out, root_out):
    h = _agg_sum(root_ref, b_ref, acc_ref, inv_ref, wrel_ref)
    h_out[...] = h
    root_out[...] = _mm(h, wrootn_ref[...])


_k_mid = pl.pallas_call(
    _k_mid_body,
    grid=(NB,),
    in_specs=[
        pl.BlockSpec((BLK, D), lambda i: (i, 0)),
        pl.BlockSpec((1, D), lambda i: (0, 0)),
        pl.BlockSpec((R, NC, BLK, D), lambda i: (0, 0, i, 0)),
        pl.BlockSpec((R, BLK, 1), lambda i: (0, i, 0)),
        pl.BlockSpec((R, D, D), lambda i: (0, 0, 0)),
        pl.BlockSpec((D, D), lambda i: (0, 0)),
    ],
    out_specs=[
        pl.BlockSpec((BLK, D), lambda i: (i, 0)),
        pl.BlockSpec((BLK, D), lambda i: (i, 0)),
    ],
    out_shape=[
        jax.ShapeDtypeStruct((N, D), jnp.float32),
        jax.ShapeDtypeStruct((N, D), jnp.float32),
    ],
)


def _k_mlp_body(root_ref, b_ref, acc_ref, inv_ref, wrel_ref,
                wo1_ref, bo1_ref, wo2_ref, bo2_ref, wo3_ref, bo3_ref,
                out_ref):
    h = _agg_sum(root_ref, b_ref, acc_ref, inv_ref, wrel_ref)
    o = jnp.maximum(_mm(h, wo1_ref[...]) + bo1_ref[0][None, :], 0.0)
    o = jnp.maximum(_mm(o, wo2_ref[...]) + bo2_ref[0][None, :], 0.0)
    out_ref[...] = _mm(o, wo3_ref[...]) + bo3_ref[0][None, :]


_k_mlp = pl.pallas_call(
    _k_mlp_body,
    grid=(NB,),
    in_specs=[
        pl.BlockSpec((BLK, D), lambda i: (i, 0)),
        pl.BlockSpec((1, D), lambda i: (0, 0)),
        pl.BlockSpec((R, NC, BLK, D), lambda i: (0, 0, i, 0)),
        pl.BlockSpec((R, BLK, 1), lambda i: (0, i, 0)),
        pl.BlockSpec((R, D, D), lambda i: (0, 0, 0)),
        pl.BlockSpec((D, 512), lambda i: (0, 0)),
        pl.BlockSpec((1, 512), lambda i: (0, 0)),
        pl.BlockSpec((512, 256), lambda i: (0, 0)),
        pl.BlockSpec((1, 256), lambda i: (0, 0)),
        pl.BlockSpec((256, 128), lambda i: (0, 0)),
        pl.BlockSpec((1, 128), lambda i: (0, 0)),
    ],
    out_specs=pl.BlockSpec((BLK, 128), lambda i: (i, 0)),
    out_shape=jax.ShapeDtypeStruct((N, 128), jnp.float32),
)


def kernel(x, edge_index, edge_type, W_in, b_in, w1_rel, w1_root, b1,
           w2_rel, w2_root, b2, w3_rel, w3_root, b3,
           Wo1, bo1, Wo2, bo2, Wo3, bo3):
    src = edge_index[0]
    dst = edge_index[1]

    cnt2, lsrc, ldst, ncnt = _pass_a(src, dst, edge_type)
    inv3 = _pass_b(cnt2)[:CNT].reshape(R, N, 1)

    h, root = _k_in(x, W_in, b_in.reshape(1, D), w1_root)
    acc = _pass_m(h, lsrc, ldst, ncnt)
    h, root = _k_mid(root, b1.reshape(1, D), acc, inv3, w1_rel, w2_root)
    acc = _pass_m(h, lsrc, ldst, ncnt)
    h, root = _k_mid(root, b2.reshape(1, D), acc, inv3, w2_rel, w3_root)
    acc = _pass_m(h, lsrc, ldst, ncnt)

    wo3p = jnp.pad(Wo3, ((0, 0), (0, 128 - Wo3.shape[1])))
    bo3p = jnp.pad(bo3, (0, 128 - bo3.shape[0]))
    out = _k_mlp(root, b3.reshape(1, D), acc, inv3, w3_rel,
                 Wo1, bo1.reshape(1, 512), Wo2, bo2.reshape(1, 256),
                 wo3p, bo3p.reshape(1, 128))
    return out[:, :Wo3.shape[1]]


# speedup vs baseline: 1.0826x; 1.0424x over previous
import jax
import jax.numpy as jnp
from jax import lax
from jax.experimental import pallas as pl
from jax.experimental.pallas import tpu as pltpu
from jax.experimental.pallas import tpu_sc as plsc

N = 10000
E = 320000
D = 128
R = 3

NC = 2
NS = 16
NW = NC * NS
EPW = E // NW
K = 128
NFULL = EPW // K
TAIL = EPW - NFULL * K
CNT = R * N
CNT_PAD = 30720
CPT = CNT_PAD // NS
CH = 40
NCHW = EPW // CH
NSLOT = 5
NZC = N // CH
CB = 320
RSEG = ((EPW + CB - 1) // CB + 1) * CB
STG = RSEG

_mesh = plsc.VectorSubcoreMesh(core_axis_name="c", subcore_axis_name="s")


def _worker():
    cid = lax.axis_index("c")
    sid = lax.axis_index("s")
    return cid, sid, sid * NC + cid


def _pass_a_body(src_hbm, dst_hbm, typ_hbm, cnt2_hbm, lsrc_hbm, ldst_hbm,
                 ncnt_hbm, src_v, dst_v, typ_v, c_v, ones_v, zb_v,
                 s16, d16, t16, c16, o16, nv_v, stg_s, stg_d, cnt_sh):
    cid, sid, wid = _worker()
    base = wid * EPW
    iota = jnp.arange(16, dtype=jnp.int32)

    @pl.loop(0, CPT // 16)
    def _zero_zb(i):
        zb_v[pl.ds(i * 16, 16)] = jnp.zeros((16,), jnp.float32)

    @pl.loop(0, K // 16)
    def _ones(i):
        ones_v[pl.ds(i * 16, 16)] = jnp.full((16,), 1.0, jnp.float32)

    o16[pl.ds(0, 16)] = jnp.full((16,), 1.0, jnp.float32)

    pltpu.sync_copy(zb_v, cnt_sh.at[pl.ds(sid * CPT, CPT)])
    plsc.subcore_barrier()

    def group(sl, sv_ref, dv_ref, tv_ref, cv_ref, offs):
        tv = tv_ref[sl]
        sv = sv_ref[sl]
        dv = dv_ref[sl]
        cv_ref[sl] = tv * N + dv
        new_offs = []
        for r in range(R):
            mi = (tv == r).astype(jnp.int32)
            pos = offs[r] + plsc.cumsum(mi) - mi
            m = mi != 0
            plsc.store_scatter(stg_s[r], [pos], sv, mask=m)
            plsc.store_scatter(stg_d[r], [pos], dv, mask=m)
            new_offs.append(offs[r] + jnp.sum(mi))
        return tuple(new_offs)

    @pl.loop(0, NFULL, init_carry=(0, 0, 0))
    def _chunks(i, offs):
        off = base + i * K
        pltpu.sync_copy(src_hbm.at[pl.ds(off, K)], src_v)
        pltpu.sync_copy(dst_hbm.at[pl.ds(off, K)], dst_v)
        pltpu.sync_copy(typ_hbm.at[pl.ds(off, K)], typ_v)
        for j in range(K // 16):
            offs = group(pl.ds(j * 16, 16), src_v, dst_v, typ_v, c_v, offs)
        pltpu.sync_copy(ones_v, cnt_sh.at[c_v], add=True)
        return offs

    toff = base + NFULL * K
    pltpu.sync_copy(src_hbm.at[pl.ds(toff, TAIL)], s16)
    pltpu.sync_copy(dst_hbm.at[pl.ds(toff, TAIL)], d16)
    pltpu.sync_copy(typ_hbm.at[pl.ds(toff, TAIL)], t16)
    offs = group(pl.ds(0, 16), s16, d16, t16, c16, _chunks)
    pltpu.sync_copy(o16, cnt_sh.at[c16], add=True)

    nchs = []
    for r in range(R):
        for kk in range(3):
            pos = offs[r] + kk * 16 + iota
            plsc.store_scatter(stg_s[r], [pos],
                               jnp.zeros((16,), jnp.int32))
            plsc.store_scatter(stg_d[r], [pos],
                               jnp.full((16,), N, jnp.int32))
        nch = (offs[r] + (CH - 1)) // CH
        nchs.append(nch)

        lbase = (r * NW + wid) * RSEG
        nbig = (offs[r] + (CB - 1)) // CB

        @pl.loop(0, nbig)
        def _dma_out(i):
            o = pl.multiple_of(i * CB, CB)
            pltpu.sync_copy(stg_s[r].at[pl.ds(o, CB)],
                            lsrc_hbm.at[pl.ds(lbase + o, CB)])
            pltpu.sync_copy(stg_d[r].at[pl.ds(o, CB)],
                            ldst_hbm.at[pl.ds(lbase + o, CB)])

    cvec = (jnp.where(iota == 0, nchs[0], 0)
            + jnp.where(iota == 1, nchs[1], 0)
            + jnp.where(iota == 2, nchs[2], 0)).astype(jnp.int32)
    nv_v[pl.ds(0, 16)] = cvec
    pltpu.sync_copy(nv_v, ncnt_hbm.at[wid, 0])

    plsc.subcore_barrier()
    pltpu.sync_copy(cnt_sh.at[pl.ds(sid * CPT, CPT)],
                    cnt2_hbm.at[cid, 0, pl.ds(sid * CPT, CPT)])


_pass_a = pl.kernel(
    _pass_a_body,
    out_type=(
        jax.ShapeDtypeStruct((NC, 1, CNT_PAD), jnp.float32),
        jax.ShapeDtypeStruct((R * NW * RSEG,), jnp.int32),
        jax.ShapeDtypeStruct((R * NW * RSEG,), jnp.int32),
        jax.ShapeDtypeStruct((NW, 1, 16), jnp.int32),
    ),
    mesh=_mesh,
    compiler_params=pltpu.CompilerParams(needs_layout_passes=False),
    scratch_types=[
        pltpu.VMEM((K,), jnp.int32),
        pltpu.VMEM((K,), jnp.int32),
        pltpu.VMEM((K,), jnp.int32),
        pltpu.VMEM((K,), jnp.int32),
        pltpu.VMEM((K,), jnp.float32),
        pltpu.VMEM((CPT,), jnp.float32),
        pltpu.VMEM((16,), jnp.int32),
        pltpu.VMEM((16,), jnp.int32),
        pltpu.VMEM((16,), jnp.int32),
        pltpu.VMEM((16,), jnp.int32),
        pltpu.VMEM((16,), jnp.float32),
        pltpu.VMEM((16,), jnp.int32),
        [pltpu.VMEM((STG,), jnp.int32) for _ in range(R)],
        [pltpu.VMEM((STG,), jnp.int32) for _ in range(R)],
        pltpu.VMEM_SHARED((CNT_PAD,), jnp.float32),
    ],
)


IPW = CNT_PAD // NS
ZR = 80


def _pass_m_body(h_hbm, lsrc_hbm, ldst_hbm, ncnt_hbm, cnt2_hbm,
                 acc_hbm, inv_hbm,
                 gbufs, dbufs, rows, zrow, cbuf, a_v, b_v, inv_v,
                 gsems, isems, ssems, acc_sh):
    cid, sid, wid = _worker()
    iota = jnp.arange(16, dtype=jnp.int32)

    @pl.when(cid == 0)
    def _inv_table():
        off = sid * IPW
        pltpu.sync_copy(cnt2_hbm.at[0, 0, pl.ds(off, IPW)], a_v)
        pltpu.sync_copy(cnt2_hbm.at[1, 0, pl.ds(off, IPW)], b_v)

        @pl.loop(0, IPW // 16)
        def _inv(i):
            sl = pl.ds(i * 16, 16)
            inv_v[sl] = 1.0 / jnp.maximum(a_v[sl] + b_v[sl], 1.0)

        pltpu.sync_copy(inv_v, inv_hbm.at[pl.ds(off, IPW)])

    pltpu.sync_copy(ncnt_hbm.at[wid, 0], cbuf)
    nch_vec = cbuf[pl.ds(0, 16)]

    @pl.loop(0, ZR)
    def _zero_zrow(i):
        for j in range(D // 16):
            zrow[i, pl.ds(j * 16, 16)] = jnp.zeros((16,), jnp.float32)

    for r in range(R):
        nv = jnp.max(jnp.where(iota == r, nch_vec, 0))

        lbase = (r * NW + wid) * RSEG

        def idx_start(c, s):
            off = pl.ds(lbase + pl.multiple_of(c * CH, CH), CH)
            pltpu.async_copy(lsrc_hbm.at[off], gbufs[s], isems[s])
            pltpu.async_copy(ldst_hbm.at[off], dbufs[s], isems[s])

        def idx_wait(c, s):
            off = pl.ds(lbase + pl.multiple_of(c * CH, CH), CH)
            pltpu.make_async_copy(lsrc_hbm.at[off], gbufs[s],
                                  isems[s]).wait()
            pltpu.make_async_copy(ldst_hbm.at[off], dbufs[s],
                                  isems[s]).wait()

        def scat_wait(s):
            pltpu.make_async_copy(rows[s], acc_sh.at[dbufs[s]],
                                  ssems[s]).wait()

        for k in range(8):
            zc = sid + NS * k

            @pl.when(zc < N // ZR)
            def _zero_chunk():
                off = pl.multiple_of(zc * ZR, ZR)
                pltpu.sync_copy(zrow, acc_sh.at[pl.ds(off, ZR)])

        plsc.subcore_barrier()

        for c in range(3):

            @pl.when(c < nv)
            def _prime_idx():
                idx_start(c, c)

        for c in range(2):

            @pl.when(c < nv)
            def _prime_gather():
                idx_wait(c, c)
                pltpu.async_copy(h_hbm.at[gbufs[c]], rows[c], gsems[c])

        nvis = (nv + (NSLOT - 1)) // NSLOT

        @pl.loop(0, nvis)
        def _visits(i):
            for b in range(NSLOT):
                c = i * NSLOT + b

                s3 = (b + 3) % NSLOT
                c3 = c + 3

                @pl.when(c3 < nv)
                def _issue_idx():
                    @pl.when(c3 >= NSLOT)
                    def _drain_scat():
                        scat_wait(s3)

                    idx_start(c3, s3)

                s2 = (b + 2) % NSLOT
                c2 = c + 2

                @pl.when(c2 < nv)
                def _issue_gather():
                    idx_wait(c2, s2)
                    pltpu.async_copy(h_hbm.at[gbufs[s2]], rows[s2],
                                     gsems[s2])

                @pl.when(c < nv)
                def _process():
                    pltpu.make_async_copy(h_hbm.at[gbufs[b]], rows[b],
                                          gsems[b]).wait()
                    pltpu.async_copy(rows[b], acc_sh.at[dbufs[b]],
                                     ssems[b], add=True)

        for s in range(NSLOT):

            @pl.when(s < nv)
            def _final_drain():
                scat_wait(s)

        plsc.subcore_barrier()
        for k in range(8):
            zc = sid + NS * k

            @pl.when(zc < N // ZR)
            def _dump_chunk():
                off = pl.multiple_of(zc * ZR, ZR)
                pltpu.sync_copy(acc_sh.at[pl.ds(off, ZR)],
                                acc_hbm.at[r, cid, pl.ds(off, ZR)])

        plsc.subcore_barrier()


_pass_m = pl.kernel(
    _pass_m_body,
    out_type=(
        jax.ShapeDtypeStruct((R, NC, N, D), jnp.float32),
        jax.ShapeDtypeStruct((CNT_PAD,), jnp.float32),
    ),
    mesh=_mesh,
    compiler_params=pltpu.CompilerParams(needs_layout_passes=False),
    scratch_types=[
        [pltpu.VMEM((CH,), jnp.int32) for _ in range(NSLOT)],
        [pltpu.VMEM((CH,), jnp.int32) for _ in range(NSLOT)],
        [pltpu.VMEM((CH, D), jnp.float32) for _ in range(NSLOT)],
        pltpu.VMEM((ZR, D), jnp.float32),
        pltpu.VMEM((16,), jnp.int32),
        pltpu.VMEM((IPW,), jnp.float32),
        pltpu.VMEM((IPW,), jnp.float32),
        pltpu.VMEM((IPW,), jnp.float32),
        [pltpu.SemaphoreType.DMA for _ in range(NSLOT)],
        [pltpu.SemaphoreType.DMA for _ in range(NSLOT)],
        [pltpu.SemaphoreType.DMA for _ in range(NSLOT)],
        pltpu.VMEM_SHARED((N + 8, D), jnp.float32),
    ],
)


BLK = 1000
NB = N // BLK


def _mm(a, b):
    return jnp.dot(a, b, preferred_element_type=jnp.float32)


def _k_in_body(x_ref, win_ref, bin_ref, wroot_ref, h_out, root_out):
    h = jnp.maximum(_mm(x_ref[...], win_ref[...]) + bin_ref[0][None, :], 0.0)
    h_out[...] = h
    root_out[...] = _mm(h, wroot_ref[...])


_k_in = pl.pallas_call(
    _k_in_body,
    grid=(NB,),
    in_specs=[
        pl.BlockSpec((BLK, D), lambda i: (i, 0)),
        pl.BlockSpec((D, D), lambda i: (0, 0)),
        pl.BlockSpec((1, D), lambda i: (0, 0)),
        pl.BlockSpec((D, D), lambda i: (0, 0)),
    ],
    out_specs=[
        pl.BlockSpec((BLK, D), lambda i: (i, 0)),
        pl.BlockSpec((BLK, D), lambda i: (i, 0)),
    ],
    out_shape=[
        jax.ShapeDtypeStruct((N, D), jnp.float32),
        jax.ShapeDtypeStruct((N, D), jnp.float32),
    ],
)


def _agg_sum(root_ref, b_ref, acc_ref, inv_ref, wrel_ref):
    t = root_ref[...] + b_ref[0][None, :]
    for r in range(R):
        agg = (acc_ref[r, 0] + acc_ref[r, 1]) * inv_ref[r]
        t = t + _mm(agg, wrel_ref[r])
    return jnp.maximum(t, 0.0)


def _k_mid_body(root_ref, b_ref, acc_ref, inv_ref, wrel_ref, wrootn_ref,
                h_out, root_out):
    h = _agg_sum(root_ref, b_ref, acc_ref, inv_ref, wrel_ref)
    h_out[...] = h
    root_out[...] = _mm(h, wrootn_ref[...])


_k_mid = pl.pallas_call(
    _k_mid_body,
    grid=(NB,),
    in_specs=[
        pl.BlockSpec((BLK, D), lambda i: (i, 0)),
        pl.BlockSpec((1, D), lambda i: (0, 0)),
        pl.BlockSpec((R, NC, BLK, D), lambda i: (0, 0, i, 0)),
        pl.BlockSpec((R, BLK, 1), lambda i: (0, i, 0)),
        pl.BlockSpec((R, D, D), lambda i: (0, 0, 0)),
        pl.BlockSpec((D, D), lambda i: (0, 0)),
    ],
    out_specs=[
        pl.BlockSpec((BLK, D), lambda i: (i, 0)),
        pl.BlockSpec((BLK, D), lambda i: (i, 0)),
    ],
    out_shape=[
        jax.ShapeDtypeStruct((N, D), jnp.float32),
        jax.ShapeDtypeStruct((N, D), jnp.float32),
    ],
)


def _k_mlp_body(root_ref, b_ref, acc_ref, inv_ref, wrel_ref,
                wo1_ref, bo1_ref, wo2_ref, bo2_ref, wo3_ref, bo3_ref,
                out_ref):
    h = _agg_sum(root_ref, b_ref, acc_ref, inv_ref, wrel_ref)
    o = jnp.maximum(_mm(h, wo1_ref[...]) + bo1_ref[0][None, :], 0.0)
    o = jnp.maximum(_mm(o, wo2_ref[...]) + bo2_ref[0][None, :], 0.0)
    out_ref[...] = _mm(o, wo3_ref[...]) + bo3_ref[0][None, :]


_k_mlp = pl.pallas_call(
    _k_mlp_body,
    grid=(NB,),
    in_specs=[
        pl.BlockSpec((BLK, D), lambda i: (i, 0)),
        pl.BlockSpec((1, D), lambda i: (0, 0)),
        pl.BlockSpec((R, NC, BLK, D), lambda i: (0, 0, i, 0)),
        pl.BlockSpec((R, BLK, 1), lambda i: (0, i, 0)),
        pl.BlockSpec((R, D, D), lambda i: (0, 0, 0)),
        pl.BlockSpec((D, 512), lambda i: (0, 0)),
        pl.BlockSpec((1, 512), lambda i: (0, 0)),
        pl.BlockSpec((512, 256), lambda i: (0, 0)),
        pl.BlockSpec((1, 256), lambda i: (0, 0)),
        pl.BlockSpec((256, 128), lambda i: (0, 0)),
        pl.BlockSpec((1, 128), lambda i: (0, 0)),
    ],
    out_specs=pl.BlockSpec((BLK, 128), lambda i: (i, 0)),
    out_shape=jax.ShapeDtypeStruct((N, 128), jnp.float32),
)


def kernel(x, edge_index, edge_type, W_in, b_in, w1_rel, w1_root, b1,
           w2_rel, w2_root, b2, w3_rel, w3_root, b3,
           Wo1, bo1, Wo2, bo2, Wo3, bo3):
    src = edge_index[0]
    dst = edge_index[1]

    cnt2, lsrc, ldst, ncnt = _pass_a(src, dst, edge_type)

    h, root = _k_in(x, W_in, b_in.reshape(1, D), w1_root)
    acc, invf = _pass_m(h, lsrc, ldst, ncnt, cnt2)
    inv3 = invf[:CNT].reshape(R, N, 1)
    h, root = _k_mid(root, b1.reshape(1, D), acc, inv3, w1_rel, w2_root)
    acc, _ = _pass_m(h, lsrc, ldst, ncnt, cnt2)
    h, root = _k_mid(root, b2.reshape(1, D), acc, inv3, w2_rel, w3_root)
    acc, _ = _pass_m(h, lsrc, ldst, ncnt, cnt2)

    wo3p = jnp.pad(Wo3, ((0, 0), (0, 128 - Wo3.shape[1])))
    bo3p = jnp.pad(bo3, (0, 128 - bo3.shape[0]))
    out = _k_mlp(root, b3.reshape(1, D), acc, inv3, w3_rel,
                 Wo1, bo1.reshape(1, 512), Wo2, bo2.reshape(1, 256),
                 wo3p, bo3p.reshape(1, 128))
    return out[:, :Wo3.shape[1]]


# speedup vs baseline: 1.2296x; 1.1358x over previous
import jax
import jax.numpy as jnp
from jax import lax
from jax.experimental import pallas as pl
from jax.experimental.pallas import tpu as pltpu
from jax.experimental.pallas import tpu_sc as plsc

N = 10000
E = 320000
D = 128
R = 3

NC = 2
NS = 16
NW = NC * NS
EPW = E // NW
KA = 2000
KA_CH = EPW // KA
CNT = R * N
CNT_PAD = 30720
CPT = CNT_PAD // NS
CH = 40
NCHW = EPW // CH
NSLOT = 5
NZC = N // CH
CB = 320
RSEG = ((EPW + CB - 1) // CB + 1) * CB
STG = RSEG

_mesh = plsc.VectorSubcoreMesh(core_axis_name="c", subcore_axis_name="s")


def _worker():
    cid = lax.axis_index("c")
    sid = lax.axis_index("s")
    return cid, sid, sid * NC + cid


def _pass_a_body(src_hbm, dst_hbm, typ_hbm, cnt2_hbm, lsrc_hbm, ldst_hbm,
                 ncnt_hbm, src_v, dst_v, typ_v, ones_v, zb_v,
                 nv_v, csets, csems, stg_s, stg_d, cnt_sh):
    cid, sid, wid = _worker()
    base = wid * EPW
    iota = jnp.arange(16, dtype=jnp.int32)

    @pl.loop(0, CPT // 16)
    def _zero_zb(i):
        zb_v[pl.ds(i * 16, 16)] = jnp.zeros((16,), jnp.float32)

    @pl.loop(0, 128 // 16)
    def _ones(i):
        ones_v[pl.ds(i * 16, 16)] = jnp.full((16,), 1.0, jnp.float32)

    for s in range(2):
        for kk in range(3):
            csets[s][15, 0, pl.ds(80 + kk * 16, 16)] = jnp.full(
                (16,), CNT_PAD - 16, jnp.int32)

    pltpu.sync_copy(zb_v, cnt_sh.at[pl.ds(sid * CPT, CPT)])
    plsc.subcore_barrier()

    def drain_counts(s):
        for q in range(16):
            pltpu.make_async_copy(ones_v, cnt_sh.at[csets[s].at[q, 0]],
                                  csems[s]).wait()

    offs = (0, 0, 0)
    for i in range(KA_CH):
        s = i % 2
        off = base + i * KA
        pltpu.sync_copy(src_hbm.at[pl.ds(off, KA)], src_v)
        pltpu.sync_copy(dst_hbm.at[pl.ds(off, KA)], dst_v)
        pltpu.sync_copy(typ_hbm.at[pl.ds(off, KA)], typ_v)
        if i >= 2:
            drain_counts(s)

        @pl.loop(0, KA // 16, init_carry=offs)
        def _groups(j, offs):
            sl = pl.ds(pl.multiple_of(j * 16, 16), 16)
            tv = typ_v[sl]
            sv = src_v[sl]
            dv = dst_v[sl]
            q = j // 8
            slot = j - q * 8
            cs = pl.ds(pl.multiple_of(slot * 16, 16), 16)
            csets[s][q, 0, cs] = tv * N + dv
            new_offs = []
            for r in range(R):
                mi = (tv == r).astype(jnp.int32)
                pos = offs[r] + plsc.cumsum(mi) - mi
                m = mi != 0
                plsc.store_scatter(stg_s[r], [pos], sv, mask=m)
                plsc.store_scatter(stg_d[r], [pos], dv, mask=m)
                new_offs.append(offs[r] + jnp.sum(mi))
            return tuple(new_offs)

        offs = _groups
        for q in range(16):
            pltpu.async_copy(ones_v, cnt_sh.at[csets[s].at[q, 0]], csems[s],
                             add=True)

    drain_counts(1)
    drain_counts(0)

    nchs = []
    for r in range(R):
        for kk in range(3):
            pos = offs[r] + kk * 16 + iota
            plsc.store_scatter(stg_s[r], [pos],
                               jnp.zeros((16,), jnp.int32))
            plsc.store_scatter(stg_d[r], [pos],
                               jnp.full((16,), N, jnp.int32))
        nch = (offs[r] + (CH - 1)) // CH
        nchs.append(nch)

        lbase = (r * NW + wid) * RSEG
        nbig = (offs[r] + (CB - 1)) // CB

        @pl.loop(0, nbig)
        def _dma_out(i):
            o = pl.multiple_of(i * CB, CB)
            pltpu.sync_copy(stg_s[r].at[pl.ds(o, CB)],
                            lsrc_hbm.at[pl.ds(lbase + o, CB)])
            pltpu.sync_copy(stg_d[r].at[pl.ds(o, CB)],
                            ldst_hbm.at[pl.ds(lbase + o, CB)])

    cvec = (jnp.where(iota == 0, nchs[0], 0)
            + jnp.where(iota == 1, nchs[1], 0)
            + jnp.where(iota == 2, nchs[2], 0)).astype(jnp.int32)
    nv_v[pl.ds(0, 16)] = cvec
    pltpu.sync_copy(nv_v, ncnt_hbm.at[wid, 0])

    plsc.subcore_barrier()
    pltpu.sync_copy(cnt_sh.at[pl.ds(sid * CPT, CPT)],
                    cnt2_hbm.at[cid, 0, pl.ds(sid * CPT, CPT)])


_pass_a = pl.kernel(
    _pass_a_body,
    out_type=(
        jax.ShapeDtypeStruct((NC, 1, CNT_PAD), jnp.float32),
        jax.ShapeDtypeStruct((R * NW * RSEG,), jnp.int32),
        jax.ShapeDtypeStruct((R * NW * RSEG,), jnp.int32),
        jax.ShapeDtypeStruct((NW, 1, 16), jnp.int32),
    ),
    mesh=_mesh,
    compiler_params=pltpu.CompilerParams(needs_layout_passes=False),
    scratch_types=[
        pltpu.VMEM((KA,), jnp.int32),
        pltpu.VMEM((KA,), jnp.int32),
        pltpu.VMEM((KA,), jnp.int32),
        pltpu.VMEM((128,), jnp.float32),
        pltpu.VMEM((CPT,), jnp.float32),
        pltpu.VMEM((16,), jnp.int32),
        [pltpu.VMEM((16, 1, 128), jnp.int32) for _ in range(2)],
        [pltpu.SemaphoreType.DMA for _ in range(2)],
        [pltpu.VMEM((STG,), jnp.int32) for _ in range(R)],
        [pltpu.VMEM((STG,), jnp.int32) for _ in range(R)],
        pltpu.VMEM_SHARED((CNT_PAD,), jnp.float32),
    ],
)


IPW = CNT_PAD // NS
ZR = 80


def _pass_m_body(h_hbm, lsrc_hbm, ldst_hbm, ncnt_hbm, cnt2_hbm,
                 acc_hbm, inv_hbm,
                 gbufs, dbufs, rows, zrow, cbuf, a_v, b_v, inv_v,
                 gsems, isems, ssems, acc_sh):
    cid, sid, wid = _worker()
    iota = jnp.arange(16, dtype=jnp.int32)

    @pl.when(cid == 0)
    def _inv_table():
        off = sid * IPW
        pltpu.sync_copy(cnt2_hbm.at[0, 0, pl.ds(off, IPW)], a_v)
        pltpu.sync_copy(cnt2_hbm.at[1, 0, pl.ds(off, IPW)], b_v)

        @pl.loop(0, IPW // 16)
        def _inv(i):
            sl = pl.ds(i * 16, 16)
            inv_v[sl] = 1.0 / jnp.maximum(a_v[sl] + b_v[sl], 1.0)

        pltpu.sync_copy(inv_v, inv_hbm.at[pl.ds(off, IPW)])

    pltpu.sync_copy(ncnt_hbm.at[wid, 0], cbuf)
    nch_vec = cbuf[pl.ds(0, 16)]

    @pl.loop(0, ZR)
    def _zero_zrow(i):
        for j in range(D // 16):
            zrow[i, pl.ds(j * 16, 16)] = jnp.zeros((16,), jnp.float32)

    for r in range(R):
        nv = jnp.max(jnp.where(iota == r, nch_vec, 0))

        lbase = (r * NW + wid) * RSEG

        def idx_start(c, s):
            off = pl.ds(lbase + pl.multiple_of(c * CH, CH), CH)
            pltpu.async_copy(lsrc_hbm.at[off], gbufs[s], isems[s])
            pltpu.async_copy(ldst_hbm.at[off], dbufs[s], isems[s])

        def idx_wait(c, s):
            off = pl.ds(lbase + pl.multiple_of(c * CH, CH), CH)
            pltpu.make_async_copy(lsrc_hbm.at[off], gbufs[s],
                                  isems[s]).wait()
            pltpu.make_async_copy(ldst_hbm.at[off], dbufs[s],
                                  isems[s]).wait()

        def scat_wait(s):
            pltpu.make_async_copy(rows[s], acc_sh.at[dbufs[s]],
                                  ssems[s]).wait()

        for k in range(8):
            zc = sid + NS * k

            @pl.when(zc < N // ZR)
            def _zero_chunk():
                off = pl.multiple_of(zc * ZR, ZR)
                pltpu.sync_copy(zrow, acc_sh.at[pl.ds(off, ZR)])

        plsc.subcore_barrier()

        for c in range(3):

            @pl.when(c < nv)
            def _prime_idx():
                idx_start(c, c)

        for c in range(2):

            @pl.when(c < nv)
            def _prime_gather():
                idx_wait(c, c)
                pltpu.async_copy(h_hbm.at[gbufs[c]], rows[c], gsems[c])

        nvis = (nv + (NSLOT - 1)) // NSLOT

        @pl.loop(0, nvis)
        def _visits(i):
            for b in range(NSLOT):
                c = i * NSLOT + b

                s3 = (b + 3) % NSLOT
                c3 = c + 3

                @pl.when(c3 < nv)
                def _issue_idx():
                    @pl.when(c3 >= NSLOT)
                    def _drain_scat():
                        scat_wait(s3)

                    idx_start(c3, s3)

                s2 = (b + 2) % NSLOT
                c2 = c + 2

                @pl.when(c2 < nv)
                def _issue_gather():
                    idx_wait(c2, s2)
                    pltpu.async_copy(h_hbm.at[gbufs[s2]], rows[s2],
                                     gsems[s2])

                @pl.when(c < nv)
                def _process():
                    pltpu.make_async_copy(h_hbm.at[gbufs[b]], rows[b],
                                          gsems[b]).wait()
                    pltpu.async_copy(rows[b], acc_sh.at[dbufs[b]],
                                     ssems[b], add=True)

        for s in range(NSLOT):

            @pl.when(s < nv)
            def _final_drain():
                scat_wait(s)

        plsc.subcore_barrier()
        for k in range(8):
            zc = sid + NS * k

            @pl.when(zc < N // ZR)
            def _dump_chunk():
                off = pl.multiple_of(zc * ZR, ZR)
                pltpu.sync_copy(acc_sh.at[pl.ds(off, ZR)],
                                acc_hbm.at[r, cid, pl.ds(off, ZR)])

        plsc.subcore_barrier()


_pass_m = pl.kernel(
    _pass_m_body,
    out_type=(
        jax.ShapeDtypeStruct((R, NC, N, D), jnp.float32),
        jax.ShapeDtypeStruct((CNT_PAD,), jnp.float32),
    ),
    mesh=_mesh,
    compiler_params=pltpu.CompilerParams(needs_layout_passes=False),
    scratch_types=[
        [pltpu.VMEM((CH,), jnp.int32) for _ in range(NSLOT)],
        [pltpu.VMEM((CH,), jnp.int32) for _ in range(NSLOT)],
        [pltpu.VMEM((CH, D), jnp.float32) for _ in range(NSLOT)],
        pltpu.VMEM((ZR, D), jnp.float32),
        pltpu.VMEM((16,), jnp.int32),
        pltpu.VMEM((IPW,), jnp.float32),
        pltpu.VMEM((IPW,), jnp.float32),
        pltpu.VMEM((IPW,), jnp.float32),
        [pltpu.SemaphoreType.DMA for _ in range(NSLOT)],
        [pltpu.SemaphoreType.DMA for _ in range(NSLOT)],
        [pltpu.SemaphoreType.DMA for _ in range(NSLOT)],
        pltpu.VMEM_SHARED((N + 8, D), jnp.float32),
    ],
)


BLK = 1000
NB = N // BLK


def _mm(a, b):
    return jnp.dot(a, b, preferred_element_type=jnp.float32)


def _k_in_body(x_ref, win_ref, bin_ref, wroot_ref, h_out, root_out):
    h = jnp.maximum(_mm(x_ref[...], win_ref[...]) + bin_ref[0][None, :], 0.0)
    h_out[...] = h
    root_out[...] = _mm(h, wroot_ref[...])


_k_in = pl.pallas_call(
    _k_in_body,
    grid=(NB,),
    in_specs=[
        pl.BlockSpec((BLK, D), lambda i: (i, 0)),
        pl.BlockSpec((D, D), lambda i: (0, 0)),
        pl.BlockSpec((1, D), lambda i: (0, 0)),
        pl.BlockSpec((D, D), lambda i: (0, 0)),
    ],
    out_specs=[
        pl.BlockSpec((BLK, D), lambda i: (i, 0)),
        pl.BlockSpec((BLK, D), lambda i: (i, 0)),
    ],
    out_shape=[
        jax.ShapeDtypeStruct((N, D), jnp.float32),
        jax.ShapeDtypeStruct((N, D), jnp.float32),
    ],
)


def _agg_sum(root_ref, b_ref, acc_ref, inv_ref, wrel_ref):
    t = root_ref[...] + b_ref[0][None, :]
    for r in range(R):
        agg = (acc_ref[r, 0] + acc_ref[r, 1]) * inv_ref[r]
        t = t + _mm(agg, wrel_ref[r])
    return jnp.maximum(t, 0.0)


def _k_mid_body(root_ref, b_ref, acc_ref, inv_ref, wrel_ref, wrootn_ref,
                h_out, root_out):
    h = _agg_sum(root_ref, b_ref, acc_ref, inv_ref, wrel_ref)
    h_out[...] = h
    root_out[...] = _mm(h, wrootn_ref[...])


_k_mid = pl.pallas_call(
    _k_mid_body,
    grid=(NB,),
    in_specs=[
        pl.BlockSpec((BLK, D), lambda i: (i, 0)),
        pl.BlockSpec((1, D), lambda i: (0, 0)),
        pl.BlockSpec((R, NC, BLK, D), lambda i: (0, 0, i, 0)),
        pl.BlockSpec((R, BLK, 1), lambda i: (0, i, 0)),
        pl.BlockSpec((R, D, D), lambda i: (0, 0, 0)),
        pl.BlockSpec((D, D), lambda i: (0, 0)),
    ],
    out_specs=[
        pl.BlockSpec((BLK, D), lambda i: (i, 0)),
        pl.BlockSpec((BLK, D), lambda i: (i, 0)),
    ],
    out_shape=[
        jax.ShapeDtypeStruct((N, D), jnp.float32),
        jax.ShapeDtypeStruct((N, D), jnp.float32),
    ],
)


def _k_mlp_body(root_ref, b_ref, acc_ref, inv_ref, wrel_ref,
                wo1_ref, bo1_ref, wo2_ref, bo2_ref, wo3_ref, bo3_ref,
                out_ref):
    h = _agg_sum(root_ref, b_ref, acc_ref, inv_ref, wrel_ref)
    o = jnp.maximum(_mm(h, wo1_ref[...]) + bo1_ref[0][None, :], 0.0)
    o = jnp.maximum(_mm(o, wo2_ref[...]) + bo2_ref[0][None, :], 0.0)
    out_ref[...] = _mm(o, wo3_ref[...]) + bo3_ref[0][None, :]


_k_mlp = pl.pallas_call(
    _k_mlp_body,
    grid=(NB,),
    in_specs=[
        pl.BlockSpec((BLK, D), lambda i: (i, 0)),
        pl.BlockSpec((1, D), lambda i: (0, 0)),
        pl.BlockSpec((R, NC, BLK, D), lambda i: (0, 0, i, 0)),
        pl.BlockSpec((R, BLK, 1), lambda i: (0, i, 0)),
        pl.BlockSpec((R, D, D), lambda i: (0, 0, 0)),
        pl.BlockSpec((D, 512), lambda i: (0, 0)),
        pl.BlockSpec((1, 512), lambda i: (0, 0)),
        pl.BlockSpec((512, 256), lambda i: (0, 0)),
        pl.BlockSpec((1, 256), lambda i: (0, 0)),
        pl.BlockSpec((256, 128), lambda i: (0, 0)),
        pl.BlockSpec((1, 128), lambda i: (0, 0)),
    ],
    out_specs=pl.BlockSpec((BLK, 128), lambda i: (i, 0)),
    out_shape=jax.ShapeDtypeStruct((N, 128), jnp.float32),
)


def kernel(x, edge_index, edge_type, W_in, b_in, w1_rel, w1_root, b1,
           w2_rel, w2_root, b2, w3_rel, w3_root, b3,
           Wo1, bo1, Wo2, bo2, Wo3, bo3):
    src = edge_index[0]
    dst = edge_index[1]

    cnt2, lsrc, ldst, ncnt = _pass_a(src, dst, edge_type)

    h, root = _k_in(x, W_in, b_in.reshape(1, D), w1_root)
    acc, invf = _pass_m(h, lsrc, ldst, ncnt, cnt2)
    inv3 = invf[:CNT].reshape(R, N, 1)
    h, root = _k_mid(root, b1.reshape(1, D), acc, inv3, w1_rel, w2_root)
    acc, _ = _pass_m(h, lsrc, ldst, ncnt, cnt2)
    h, root = _k_mid(root, b2.reshape(1, D), acc, inv3, w2_rel, w3_root)
    acc, _ = _pass_m(h, lsrc, ldst, ncnt, cnt2)

    wo3p = jnp.pad(Wo3, ((0, 0), (0, 128 - Wo3.shape[1])))
    bo3p = jnp.pad(bo3, (0, 128 - bo3.shape[0]))
    out = _k_mlp(root, b3.reshape(1, D), acc, inv3, w3_rel,
                 Wo1, bo1.reshape(1, 512), Wo2, bo2.reshape(1, 256),
                 wo3p, bo3p.reshape(1, 128))
    return out[:, :Wo3.shape[1]]
